# baseline (speedup 1.0000x reference)
import functools

import jax
import jax.numpy as jnp
from jax import lax
from jax.experimental import pallas as pl
from jax.experimental.pallas import tpu as pltpu
from jax.experimental.pallas import tpu_sc as plsc

BATCH = 4096
SEQ = 200
DIM = 64

_info = plsc.get_sparse_core_info()
NC, NS, NL = _info.num_cores, _info.num_subcores, _info.num_lanes
NW = NC * NS
GROUPS_PER_W = BATCH // (8 * NW)
NQ = 4
QSP = SEQ // (2 * NQ)
QCHUNK = QSP * 16

BB = 4096
KSP = 5
SP = SEQ // 2


def _sc_gather(idx_hbm, tok_hbm, out_hbm, raw_bufs, idxp_bufs, row_bufs,
               raw_sems, in_sems, out_sems, *, groups_per_w):
    GROUPS_PER_W = groups_per_w
    wid = lax.axis_index("s") * NC + lax.axis_index("c")

    lanes = lax.iota(jnp.int32, NL)
    offs_base = (lanes // 2) * SEQ + (lanes % 2)

    def raw_base(gi):
        return (wid * GROUPS_PER_W + gi) * (8 * SEQ)

    pltpu.async_copy(idx_hbm.at[pl.ds(raw_base(0), 8 * SEQ)], raw_bufs[0],
                     raw_sems[0])

    def pair(k, _):
        for par in range(2):
            gi = k * 2 + par
            gb = par
            pltpu.make_async_copy(idx_hbm.at[pl.ds(0, 8 * SEQ)], raw_bufs[gb],
                                  raw_sems[gb]).wait()

            @pl.when(gi + 1 < GROUPS_PER_W)
            def _():
                pltpu.async_copy(idx_hbm.at[pl.ds(raw_base(gi + 1), 8 * SEQ)],
                                 raw_bufs[1 - gb], raw_sems[1 - gb])

            for q in range(NQ):
                raw = raw_bufs[gb]
                idxp = idxp_bufs[q]

                @plsc.parallel_loop(0, QSP, unroll=4)
                def _(i):
                    offs = offs_base + (2 * (q * QSP) + 2 * i)
                    idxp[pl.ds(i * NL, NL)] = plsc.load_gather(raw, [offs])

            for q in range(NQ):
                @pl.when(gi >= 1)
                def _():
                    pltpu.make_async_copy(row_bufs[q],
                                          out_hbm.at[pl.ds(0, QCHUNK)],
                                          out_sems[q]).wait()

                pltpu.async_copy(tok_hbm.at[idxp_bufs[q]], row_bufs[q],
                                 in_sems[q])

            for q in range(NQ):
                c = (wid * GROUPS_PER_W + gi) * NQ + q
                pltpu.make_async_copy(tok_hbm.at[idxp_bufs[q]], row_bufs[q],
                                      in_sems[q]).wait()
                pltpu.async_copy(row_bufs[q],
                                 out_hbm.at[pl.ds(c * QCHUNK, QCHUNK)],
                                 out_sems[q])
        return 0

    lax.fori_loop(0, GROUPS_PER_W // 2, pair, 0)

    for q in range(NQ):
        pltpu.make_async_copy(row_bufs[q], out_hbm.at[pl.ds(0, QCHUNK)],
                              out_sems[q]).wait()


def _tc_relayout(rows_ref, pos_ref, out_ref):
    x4 = rows_ref[...]
    nb = x4.shape[0] * 8
    p = pos_ref[...]
    eye = (lax.broadcasted_iota(jnp.int32, (DIM, DIM), 0) ==
           lax.broadcasted_iota(jnp.int32, (DIM, DIM), 1)).astype(jnp.float32)
    for j in range(KSP):
        x = x4[:, j].reshape(nb, 128)
        for h in range(2):
            xh = x[:, h * DIM:(h + 1) * DIM]
            xt = lax.dot_general(eye, xh, (((1,), (1,)), ((), ())),
                                 preferred_element_type=jnp.float32)
            out_ref[2 * j + h] = xt + p[j, h][:, None]


NK = 2
BH = BATCH // NK


def _tc_relayout_alias(rows_ref, pos_ref, prev_ref, out_ref):
    _tc_relayout(rows_ref, pos_ref, out_ref)


@jax.jit
def kernel(inputs, token_table, position_table):
    idx_flat = inputs.reshape(-1).astype(jnp.int32)
    mesh = plsc.VectorSubcoreMesh(core_axis_name="c", subcore_axis_name="s")
    pos = position_table.reshape(SP, 2, DIM)

    halves = []
    for k in range(NK):
        gathered = pl.kernel(
            functools.partial(_sc_gather, groups_per_w=BH // (8 * NW)),
            mesh=mesh,
            out_type=jax.ShapeDtypeStruct((BH * SEQ, DIM), jnp.float32),
            scratch_types=[
                [pltpu.VMEM((8 * SEQ,), jnp.int32) for _ in range(2)],
                [pltpu.VMEM((QCHUNK,), jnp.int32) for _ in range(NQ)],
                [pltpu.VMEM((QCHUNK, DIM), jnp.float32) for _ in range(NQ)],
                [pltpu.SemaphoreType.DMA for _ in range(2)],
                [pltpu.SemaphoreType.DMA for _ in range(NQ)],
                [pltpu.SemaphoreType.DMA for _ in range(NQ)],
            ],
            compiler_params=pltpu.CompilerParams(use_tc_tiling_on_sc=False,
                                                 needs_layout_passes=False),
        )(idx_flat[k * BH * SEQ:(k + 1) * BH * SEQ], token_table)
        halves.append(gathered.reshape(BH // 8, SP, 8, 128))

    out_phys = None
    for k in range(NK):
        in_specs = [
            pl.BlockSpec((BH // 8, KSP, 8, 128), lambda i: (0, i, 0, 0)),
            pl.BlockSpec((KSP, 2, DIM), lambda i: (i, 0, 0)),
        ]
        args = [halves[k], pos]
        if k == 0:
            body = _tc_relayout
            aliases = {}
        else:
            body = _tc_relayout_alias
            in_specs.append(pl.BlockSpec(memory_space=pl.ANY))
            args.append(out_phys)
            aliases = {2: 0}
        out_phys = pl.pallas_call(
            body,
            grid=(SP // KSP,),
            in_specs=in_specs,
            out_specs=pl.BlockSpec((2 * KSP, DIM, BH),
                                   lambda i, kk=k: (i, 0, kk)),
            out_shape=jax.ShapeDtypeStruct((SEQ, DIM, BATCH), jnp.float32),
            input_output_aliases=aliases,
        )(*args)
    return out_phys.transpose(2, 0, 1)
